# baseline (speedup 1.0000x reference)
import functools

import jax
import jax.numpy as jnp
import numpy as np
from jax import lax
from jax.experimental import pallas as pl
from jax.experimental.pallas import tpu as pltpu
from jax.experimental.pallas import tpu_sc as plsc

N = 10000
E = 160000
D = 256
NG = 64
NELEM = 84
RBF = 10
VIVO = [(3, 6), (6, 9), (9, 6), (6, 3)]

VP = 16
PV = 3 * VP
FW = 384
PW = 256
NP = 10240
NH = NP // 2
NPH = 6144
CH = 128
NCHUNK = E // CH
BE = 1600
BN = 2000

_f32 = jnp.float32


def _silu(x):
    return x * jax.lax.logistic(x)


def _init_body(x_ref, i_ref, emb_ref, wn_ref, bn_ref, xs_ref, pk_ref,
               oh_ref, cnt_ref):
    pid = pl.program_id(0)
    xv = x_ref[...]
    lane128 = lax.broadcasted_iota(jnp.int32, (BN, 128), 1)
    oh_x = (xv == lane128).astype(_f32)
    emb_rows = jnp.dot(oh_x, emb_ref[...], preferred_element_type=_f32)
    xs = (jnp.dot(emb_rows, wn_ref[...], preferred_element_type=_f32)
          + bn_ref[...])
    xs_ref[...] = xs
    pk_ref[...] = xs.astype(jnp.bfloat16)
    iv = i_ref[...]
    lane64 = lax.broadcasted_iota(jnp.int32, (BN, NG), 1)
    oh = (iv == lane64).astype(_f32)
    oh_ref[...] = oh
    ones = jnp.ones((BN, 128), _f32)
    cpart = lax.dot_general(oh, ones, (((0,), (0,)), ((), ())),
                            preferred_element_type=_f32)

    @pl.when(pid == 0)
    def _():
        cnt_ref[...] = cpart

    @pl.when(pid > 0)
    def _():
        cnt_ref[...] += cpart


def _init_nodes(x2, i2, emb_pad, wn, bn):
    return pl.pallas_call(
        _init_body,
        grid=(N // BN,),
        in_specs=[
            pl.BlockSpec((BN, 1), lambda i: (i, 0)),
            pl.BlockSpec((BN, 1), lambda i: (i, 0)),
            pl.BlockSpec((128, D), lambda i: (0, 0)),
            pl.BlockSpec((D, D), lambda i: (0, 0)),
            pl.BlockSpec((1, D), lambda i: (0, 0)),
        ],
        out_specs=[
            pl.BlockSpec((BN, D), lambda i: (i, 0)),
            pl.BlockSpec((BN, D), lambda i: (i, 0)),
            pl.BlockSpec((BN, NG), lambda i: (i, 0)),
            pl.BlockSpec((NG, 128), lambda i: (0, 0)),
        ],
        out_shape=[
            jax.ShapeDtypeStruct((N, D), _f32),
            jax.ShapeDtypeStruct((N, D), jnp.bfloat16),
            jax.ShapeDtypeStruct((N, NG), _f32),
            jax.ShapeDtypeStruct((NG, 128), _f32),
        ],
    )(x2, i2, emb_pad, wn, bn)


def _msg_body(has_v, *refs):
    if has_v:
        (g_ref, e_ref, we_ref, be_ref, w1_ref, b1_ref, wg_ref, bg_ref,
         wv_ref, mlo_ref, mhi_ref, mv_ref) = refs
    else:
        (g_ref, e_ref, we_ref, be_ref, w1_ref, b1_ref, wg_ref, bg_ref,
         mlo_ref, mhi_ref, mv_ref) = refs
    gi = g_ref[...]
    lo_f = lax.bitcast_convert_type(lax.shift_left(gi, 16), _f32)
    hi_f = lax.bitcast_convert_type(
        lax.bitwise_and(gi, jnp.int32(-65536)), _f32)
    gs = jnp.concatenate([lo_f[:, :D // 2], hi_f[:, :D // 2]], axis=1)
    if has_v:
        gv = jnp.concatenate([lo_f[:, D // 2:D // 2 + PV // 2],
                              hi_f[:, D // 2:D // 2 + PV // 2]], axis=1)
    else:
        gv = None
    d = e_ref[:, 3:4]
    mu = (lax.broadcasted_iota(jnp.int32, (BE, 128), 1).astype(_f32)
          * (1.0 / (RBF - 1)))
    rbf = jnp.exp(-10.0 * (d - mu) ** 2)
    es = (jnp.dot(rbf, we_ref[...], preferred_element_type=_f32)
          + be_ref[...])
    h = gs * es
    m = _silu(jnp.dot(h, w1_ref[...], preferred_element_type=_f32)
              + b1_ref[...])
    gate = (jnp.dot(m, wg_ref[...], preferred_element_type=_f32)
            + bg_ref[...])
    ev = e_ref[...]
    mvv = jnp.concatenate(
        [gate * ev[:, c:c + 1] for c in range(3)], axis=1)
    if has_v:
        mvv = mvv + jnp.dot(gv, wv_ref[...], preferred_element_type=_f32)
    mlo_ref[...] = m[:, :128]
    mhi_ref[...] = m[:, 128:]
    mv_ref[...] = jnp.concatenate(
        [mvv, jnp.zeros((BE, 128 - PV), _f32)], axis=1)


def _messages(g, e3, we_pad, be, w1, b1, wg_pad, bg_pad, wv_bd, offb, nb):
    has_v = wv_bd is not None
    gw = PW if has_v else D // 2
    in_specs = [
        pl.BlockSpec((BE, gw), lambda i: (i, 0)),
        pl.BlockSpec((BE, 4), lambda i: (i + offb, 0)),
        pl.BlockSpec((128, D), lambda i: (0, 0)),
        pl.BlockSpec((1, D), lambda i: (0, 0)),
        pl.BlockSpec((D, D), lambda i: (0, 0)),
        pl.BlockSpec((1, D), lambda i: (0, 0)),
        pl.BlockSpec((D, VP), lambda i: (0, 0)),
        pl.BlockSpec((1, VP), lambda i: (0, 0)),
    ]
    args = [g, e3, we_pad, be, w1, b1, wg_pad, bg_pad]
    if has_v:
        in_specs.append(pl.BlockSpec((PV, PV), lambda i: (0, 0)))
        args.append(wv_bd)
    return pl.pallas_call(
        functools.partial(_msg_body, has_v),
        grid=(nb,),
        in_specs=in_specs,
        out_specs=[
            pl.BlockSpec((BE, 128), lambda i: (i, 0)),
            pl.BlockSpec((BE, 128), lambda i: (i, 0)),
            pl.BlockSpec((BE, 128), lambda i: (i, 0)),
        ],
        out_shape=[
            jax.ShapeDtypeStruct((nb * BE, 128), _f32),
            jax.ShapeDtypeStruct((nb * BE, 128), _f32),
            jax.ShapeDtypeStruct((nb * BE, 128), _f32),
        ],
    )(*args)


def _upd_body(has_v, *refs):
    if has_v:
        (alo0, alo1, ahi0, ahi1, av0, av1, av2, av3, xsv_ref, oh_ref,
         wu_ref, bu_ref, wmix_ref, wg_ref, xsv_o, pk_o, gs_o, gv_o) = refs
        xs = xsv_ref[:, :D]
        xv = xsv_ref[:, D:D + PV]
    else:
        (alo0, alo1, ahi0, ahi1, av0, av1, av2, av3, xsv_ref, oh_ref,
         wu_ref, bu_ref, wg_ref, xsv_o, pk_o, gs_o, gv_o) = refs
        xs = xsv_ref[...]
        xv = None
    pid = pl.program_id(0)
    agg = jnp.concatenate([alo0[...] + alo1[...], ahi0[...] + ahi1[...]],
                          axis=1)
    u = _silu(jnp.dot(agg, wu_ref[...], preferred_element_type=_f32)
              + bu_ref[...])
    xs_n = xs + u
    xv_n = ((av0[:, :PV] + av1[:, :PV])
            + (av2[:, :PV] + av3[:, :PV]))
    if has_v:
        xv_n = xv_n + jnp.dot(xv, wmix_ref[...], preferred_element_type=_f32)
    xsv_o[...] = jnp.concatenate(
        [xs_n, xv_n, jnp.zeros((BN, FW - D - PV), _f32)], axis=1)
    pk_o[...] = jnp.concatenate(
        [xs_n, xv_n, jnp.zeros((BN, 2 * PW - D - PV), _f32)],
        axis=1).astype(jnp.bfloat16)
    oh = oh_ref[...]
    gsp = lax.dot_general(oh, xs_n, (((0,), (0,)), ((), ())),
                          preferred_element_type=_f32)
    gvz = jnp.dot(xv_n, wg_ref[...], preferred_element_type=_f32)
    gvp = lax.dot_general(oh, gvz, (((0,), (0,)), ((), ())),
                          preferred_element_type=_f32)

    @pl.when(pid == 0)
    def _():
        gs_o[...] = gsp
        gv_o[...] = gvp

    @pl.when(pid > 0)
    def _():
        gs_o[...] += gsp
        gv_o[...] += gvp


def _update(aggs, xsv, oh, wu, bu, wmix_bd, wg_bd):
    has_v = wmix_bd is not None
    xw = FW if has_v else D
    in_specs = [pl.BlockSpec((BN, 128), lambda i: (i, 0))
                for _ in range(8)]
    in_specs += [
        pl.BlockSpec((BN, xw), lambda i: (i, 0)),
        pl.BlockSpec((BN, NG), lambda i: (i, 0)),
        pl.BlockSpec((D, D), lambda i: (0, 0)),
        pl.BlockSpec((1, D), lambda i: (0, 0)),
    ]
    args = list(aggs) + [xsv, oh, wu, bu]
    if has_v:
        in_specs.append(pl.BlockSpec((PV, PV), lambda i: (0, 0)))
        args.append(wmix_bd)
    in_specs.append(pl.BlockSpec((PV, PV), lambda i: (0, 0)))
    args.append(wg_bd)
    return pl.pallas_call(
        functools.partial(_upd_body, has_v),
        grid=(N // BN,),
        in_specs=in_specs,
        out_specs=[
            pl.BlockSpec((BN, FW), lambda i: (i, 0)),
            pl.BlockSpec((BN, 2 * PW), lambda i: (i, 0)),
            pl.BlockSpec((NG, D), lambda i: (0, 0)),
            pl.BlockSpec((NG, PV), lambda i: (0, 0)),
        ],
        out_shape=[
            jax.ShapeDtypeStruct((N, FW), _f32),
            jax.ShapeDtypeStruct((N, 2 * PW), jnp.bfloat16),
            jax.ShapeDtypeStruct((NG, D), _f32),
            jax.ShapeDtypeStruct((NG, PV), _f32),
        ],
    )(*args)


def _final_body(*refs):
    cnt_ref = refs[0]
    gs_refs = refs[1:5]
    gv_refs = refs[5:9]
    ws_refs = refs[9:13]
    bs_refs = refs[13:17]
    us_ref, uv_ref = refs[17], refs[18]
    inv = 1.0 / jnp.maximum(cnt_ref[...][:, 0:1], 1.0)
    us = jnp.zeros((NG, 128), _f32)
    uv = jnp.zeros((NG, PV), _f32)
    for l in range(4):
        gs = gs_refs[l][...] * inv
        us = us + (jnp.dot(gs, ws_refs[l][...], preferred_element_type=_f32)
                   + bs_refs[l][...])
        uv = uv + gv_refs[l][...] * inv
    us_ref[...] = us
    uv_ref[...] = uv


def _finalize(cnt, gs_l, gv_l, ws_l, bs_l):
    return pl.pallas_call(
        _final_body,
        out_shape=[
            jax.ShapeDtypeStruct((NG, 128), _f32),
            jax.ShapeDtypeStruct((NG, PV), _f32),
        ],
    )(cnt, *gs_l, *gv_l, *ws_l, *bs_l)


@functools.cache
def _sc_mesh():
    return plsc.VectorSubcoreMesh(core_axis_name="c", subcore_axis_name="s")


def _sc_gather(table, src, width, q0, nq):
    nt0 = (nq + 31) // 32
    NT = nt0 + (nt0 % 2)

    @functools.partial(
        pl.kernel,
        out_type=jax.ShapeDtypeStruct((nq * CH, width), jnp.int32),
        mesh=_sc_mesh(),
        scratch_types=[pltpu.VMEM((CH,), jnp.int32),
                       pltpu.VMEM((CH,), jnp.int32),
                       pltpu.VMEM((CH, width), jnp.int32),
                       pltpu.VMEM((CH, width), jnp.int32),
                       pltpu.SemaphoreType.DMA,
                       pltpu.SemaphoreType.DMA,
                       pltpu.SemaphoreType.DMA,
                       pltpu.SemaphoreType.DMA],
    )
    def k(tab_h, src_h, out_h, idx0, idx1, rows0, rows1, g0, g1, w0, w1):
        wid = lax.axis_index("s") * 2 + lax.axis_index("c")

        def b_of(t):
            return jnp.minimum(wid + t * 32, nq - 1) * CH

        def start_gather(t, idx, rows, gsem):
            pltpu.sync_copy(src_h.at[pl.ds(q0 * CH + b_of(t), CH)], idx)
            pltpu.async_copy(tab_h.at[idx], rows, gsem)

        start_gather(0, idx0, rows0, g0)
        start_gather(1, idx1, rows1, g1)

        @pl.loop(0, NT, step=2)
        def _(t):
            for off, idx, rows, gsem, wsem in ((0, idx0, rows0, g0, w0),
                                               (1, idx1, rows1, g1, w1)):
                tt = t + off
                b = b_of(tt)
                pltpu.make_async_copy(tab_h.at[idx], rows, gsem).wait()
                pltpu.async_copy(rows, out_h.at[pl.ds(b, CH)], wsem)
            for off, idx, rows, gsem, wsem in ((0, idx0, rows0, g0, w0),
                                               (1, idx1, rows1, g1, w1)):
                tt = t + off
                pltpu.make_async_copy(
                    rows, out_h.at[pl.ds(b_of(tt), CH)], wsem).wait()

                @pl.when(tt + 2 < NT)
                def _():
                    start_gather(tt + 2, idx, rows, gsem)

    return k(table, src)


def _sc_scatter_m(mlo, mhi, dst, q0, nq):
    nt0 = (nq + 15) // 16
    NT = nt0 + (nt0 % 2)

    @functools.partial(
        pl.kernel,
        out_type=(jax.ShapeDtypeStruct((NP, 128), _f32),
                  jax.ShapeDtypeStruct((NP, 128), _f32)),
        mesh=_sc_mesh(),
        scratch_types=[pltpu.VMEM((CH,), jnp.int32),
                       pltpu.VMEM((CH,), jnp.int32),
                       pltpu.VMEM((CH, 128), _f32),
                       pltpu.VMEM((CH, 128), _f32),
                       pltpu.VMEM_SHARED((NP, 128), _f32),
                       pltpu.SemaphoreType.DMA,
                       pltpu.SemaphoreType.DMA,
                       pltpu.SemaphoreType.DMA,
                       pltpu.SemaphoreType.DMA],
    )
    def k(mlo_h, mhi_h, dst_h, alo_h, ahi_h,
          idx0, idx1, rows0, rows1, acc_s, l0, l1, s0, s1):
        c = lax.axis_index("c")
        s = lax.axis_index("s")

        @pl.loop(0, CH)
        def _(r):
            @pl.loop(0, 128, step=16)
            def _(l):
                rows0[r, pl.ds(l, 16)] = jnp.zeros((16,), _f32)

        row0 = s * (NP // 16)

        @pl.loop(0, (NP // 16) // CH)
        def _(z):
            pltpu.sync_copy(rows0, acc_s.at[pl.ds(row0 + z * CH, CH)])

        plsc.subcore_barrier()

        def prep_and_load(t, idx, rows, lsem):
            q = s + t * 16
            b = jnp.minimum(q, nq - 1) * CH
            pltpu.sync_copy(dst_h.at[pl.ds(q0 * CH + b, CH)], idx)

            @pl.when(q >= nq)
            def _():
                @pl.loop(0, CH, step=16)
                def _(j):
                    idx[pl.ds(j, 16)] = jnp.full((16,), N, jnp.int32)

            @pl.when(c == 0)
            def _():
                pltpu.sync_copy(mlo_h.at[pl.ds(b, CH)], rows)

            @pl.when(c == 1)
            def _():
                pltpu.sync_copy(mhi_h.at[pl.ds(b, CH)], rows)

        prep_and_load(0, idx0, rows0, l0)
        prep_and_load(1, idx1, rows1, l1)

        @pl.loop(0, NT, step=2)
        def _(t):
            pltpu.async_copy(rows0, acc_s.at[idx0], s0, add=True)
            pltpu.async_copy(rows1, acc_s.at[idx1], s1, add=True)
            for off, idx, rows, lsem, ssem in ((0, idx0, rows0, l0, s0),
                                               (1, idx1, rows1, l1, s1)):
                tt = t + off
                pltpu.make_async_copy(rows, acc_s.at[idx], ssem).wait()

                @pl.when(tt + 2 < NT)
                def _():
                    prep_and_load(tt + 2, idx, rows, lsem)

        plsc.subcore_barrier()

        @pl.loop(0, (NP // 16) // CH)
        def _(z):
            r0 = row0 + z * CH

            @pl.when(c == 0)
            def _():
                pltpu.sync_copy(acc_s.at[pl.ds(r0, CH)],
                                alo_h.at[pl.ds(r0, CH)])

            @pl.when(c == 1)
            def _():
                pltpu.sync_copy(acc_s.at[pl.ds(r0, CH)],
                                ahi_h.at[pl.ds(r0, CH)])

    return k(mlo, mhi, dst)


def _sc_scatter_v(mv, dst, q0, nq):
    nqc = (nq + 1) // 2
    ntc0 = (nqc + 15) // 16
    NTC = ntc0 + (ntc0 % 2)

    @functools.partial(
        pl.kernel,
        out_type=(jax.ShapeDtypeStruct((NP, 128), _f32),
                  jax.ShapeDtypeStruct((NP, 128), _f32)),
        mesh=_sc_mesh(),
        scratch_types=[pltpu.VMEM((CH,), jnp.int32),
                       pltpu.VMEM((CH,), jnp.int32),
                       pltpu.VMEM((CH, 128), _f32),
                       pltpu.VMEM((CH, 128), _f32),
                       pltpu.VMEM_SHARED((NP, 128), _f32),
                       pltpu.SemaphoreType.DMA,
                       pltpu.SemaphoreType.DMA,
                       pltpu.SemaphoreType.DMA,
                       pltpu.SemaphoreType.DMA],
    )
    def k(mv_h, dst_h, av0_h, av1_h,
          idx0, idx1, rows0, rows1, acc_s, l0, l1, s0, s1):
        c = lax.axis_index("c")
        s = lax.axis_index("s")

        @pl.loop(0, CH)
        def _(r):
            @pl.loop(0, 128, step=16)
            def _(l):
                rows0[r, pl.ds(l, 16)] = jnp.zeros((16,), _f32)

        row0 = s * (NP // 16)

        @pl.loop(0, (NP // 16) // CH)
        def _(z):
            pltpu.sync_copy(rows0, acc_s.at[pl.ds(row0 + z * CH, CH)])

        plsc.subcore_barrier()

        qbase = c * nqc
        qend = jnp.minimum(qbase + nqc, nq)

        def prep_and_load(t, idx, rows, lsem):
            q = qbase + s + t * 16
            b = jnp.minimum(q, qend - 1) * CH
            pltpu.sync_copy(dst_h.at[pl.ds(q0 * CH + b, CH)], idx)

            @pl.when(q >= qend)
            def _():
                @pl.loop(0, CH, step=16)
                def _(j):
                    idx[pl.ds(j, 16)] = jnp.full((16,), N, jnp.int32)

            pltpu.sync_copy(mv_h.at[pl.ds(b, CH)], rows)

        prep_and_load(0, idx0, rows0, l0)
        prep_and_load(1, idx1, rows1, l1)

        @pl.loop(0, NTC, step=2)
        def _(t):
            pltpu.async_copy(rows0, acc_s.at[idx0], s0, add=True)
            pltpu.async_copy(rows1, acc_s.at[idx1], s1, add=True)
            for off, idx, rows, lsem, ssem in ((0, idx0, rows0, l0, s0),
                                               (1, idx1, rows1, l1, s1)):
                tt = t + off
                pltpu.make_async_copy(rows, acc_s.at[idx], ssem).wait()

                @pl.when(tt + 2 < NTC)
                def _():
                    prep_and_load(tt + 2, idx, rows, lsem)

        plsc.subcore_barrier()

        @pl.loop(0, (NP // 16) // CH)
        def _(z):
            r0 = row0 + z * CH

            @pl.when(c == 0)
            def _():
                pltpu.sync_copy(acc_s.at[pl.ds(r0, CH)],
                                av0_h.at[pl.ds(r0, CH)])

            @pl.when(c == 1)
            def _():
                pltpu.sync_copy(acc_s.at[pl.ds(r0, CH)],
                                av1_h.at[pl.ds(r0, CH)])

    return k(mv, dst)


def _block_diag(w, vi, vo):
    out = jnp.zeros((PV, PV), _f32)
    for ci in range(3):
        out = out.at[ci * VP:ci * VP + vi, ci * VP:ci * VP + vo].set(w)
    return out


def kernel(x, a, e, i, params):
    src, dst = a[0], a[1]

    we_pad = jnp.zeros((128, D), _f32).at[:RBF].set(params["dense_e"]["W"])
    be = params["dense_e"]["b"].reshape(1, D)
    emb_pad = jnp.zeros((128, D), _f32).at[:NELEM].set(params["emb"])
    wn = params["dense_n"]["W"]
    bn = params["dense_n"]["b"].reshape(1, D)

    x_s, x_pk, oh, cnt = _init_nodes(x, i.reshape(N, 1).astype(jnp.int32),
                                     emb_pad, wn, bn)

    qperm = np.concatenate([np.arange(0, D, 2), np.arange(1, D, 2)])
    rperm = np.concatenate([np.arange(0, PV, 2), np.arange(1, PV, 2)])

    xsv = x_s
    tbl = lax.bitcast_convert_type(
        x_pk.reshape(N, D // 2, 2), jnp.int32)
    gs_l, gv_l, ws_l, bs_l = [], [], [], []
    for li, ((vi, vo), lp, gp) in enumerate(
            zip(VIVO, params["mpnn"], params["glob"])):
        w1_q = lp["W1"]["W"][qperm, :]
        wg_pad = jnp.zeros((D, VP), _f32).at[:, :vo].set(lp["Wg"]["W"])
        bg_pad = jnp.zeros((1, VP), _f32).at[0, :vo].set(lp["Wg"]["b"])
        wv_bd = (_block_diag(lp["Wv"], vi, vo)[rperm, :]
                 if li > 0 else None)
        wmix_bd = _block_diag(lp["Wmix"], vi, vo) if li > 0 else None
        wgg_bd = _block_diag(gp["Wg"], vo, 3)
        ws_pad = jnp.zeros((D, 128), _f32).at[:, :3].set(gp["Ws"]["W"])
        bs_pad = jnp.zeros((1, 128), _f32).at[0, :3].set(gp["Ws"]["b"])

        we_q = we_pad[:, qperm]
        be_q = be[:, qperm]
        width = D // 2 if li == 0 else PW
        hq = NCHUNK // 2
        hb = (hq * CH) // BE
        aggs = []
        for h in range(2):
            g = _sc_gather(tbl, src, width, h * hq, hq)
            m_lo, m_hi, m_v = _messages(g, e, we_q, be_q,
                                        w1_q,
                                        lp["W1"]["b"].reshape(1, D),
                                        wg_pad, bg_pad, wv_bd,
                                        h * hb, hb)
            a_lo, a_hi = _sc_scatter_m(m_lo, m_hi, dst, h * hq, hq)
            a_v0, a_v1 = _sc_scatter_v(m_v, dst, h * hq, hq)
            aggs.append((a_lo[:N], a_hi[:N], a_v0[:N], a_v1[:N]))

        (al0, ah0, v00, v01), (al1, ah1, v10, v11) = aggs
        xsv, x_pk, gs, gv = _update((al0, al1, ah0, ah1, v00, v01, v10, v11),
                                    xsv, oh,
                                    lp["Wu"]["W"],
                                    lp["Wu"]["b"].reshape(1, D),
                                    wmix_bd, wgg_bd)
        tbl = lax.bitcast_convert_type(
            x_pk.reshape(N, PW, 2), jnp.int32)
        gs_l.append(gs)
        gv_l.append(gv)
        ws_l.append(ws_pad)
        bs_l.append(bs_pad)

    us, uv = _finalize(cnt, gs_l, gv_l, ws_l, bs_l)
    u_s = us[:, :3]
    u_v = uv.reshape(NG, 3, VP)[:, :, :3]
    return jnp.concatenate([u_s[:, :, None], u_v], axis=-1)


# speedup vs baseline: 1.1662x; 1.1662x over previous
import functools

import jax
import jax.numpy as jnp
import numpy as np
from jax import lax
from jax.experimental import pallas as pl
from jax.experimental.pallas import tpu as pltpu
from jax.experimental.pallas import tpu_sc as plsc

N = 10000
E = 160000
D = 256
NG = 64
NELEM = 84
RBF = 10
VIVO = [(3, 6), (6, 9), (9, 6), (6, 3)]

VP = 16
PV = 3 * VP
FW = 384
PW = 256
NP = 10240
NH = NP // 2
NPH = 6144
CH = 128
NCHUNK = E // CH
BE = 1600
BN = 1000

_f32 = jnp.float32


def _silu(x):
    return x * jax.lax.logistic(x)


def _init_body(x_ref, i_ref, emb_ref, wn_ref, bn_ref, xs_ref, oh_ref, cnt_ref):
    pid = pl.program_id(0)
    xv = x_ref[...]
    lane128 = lax.broadcasted_iota(jnp.int32, (BN, 128), 1)
    oh_x = (xv == lane128).astype(_f32)
    emb_rows = jnp.dot(oh_x, emb_ref[...], preferred_element_type=_f32)
    xs_ref[...] = (jnp.dot(emb_rows, wn_ref[...], preferred_element_type=_f32)
                   + bn_ref[...])
    iv = i_ref[...]
    lane64 = lax.broadcasted_iota(jnp.int32, (BN, NG), 1)
    oh = (iv == lane64).astype(_f32)
    oh_ref[...] = oh
    ones = jnp.ones((BN, 128), _f32)
    cpart = lax.dot_general(oh, ones, (((0,), (0,)), ((), ())),
                            preferred_element_type=_f32)

    @pl.when(pid == 0)
    def _():
        cnt_ref[...] = cpart

    @pl.when(pid > 0)
    def _():
        cnt_ref[...] += cpart


def _init_nodes(x2, i2, emb_pad, wn, bn):
    return pl.pallas_call(
        _init_body,
        grid=(N // BN,),
        in_specs=[
            pl.BlockSpec((BN, 1), lambda i: (i, 0)),
            pl.BlockSpec((BN, 1), lambda i: (i, 0)),
            pl.BlockSpec((128, D), lambda i: (0, 0)),
            pl.BlockSpec((D, D), lambda i: (0, 0)),
            pl.BlockSpec((1, D), lambda i: (0, 0)),
        ],
        out_specs=[
            pl.BlockSpec((BN, D), lambda i: (i, 0)),
            pl.BlockSpec((BN, NG), lambda i: (i, 0)),
            pl.BlockSpec((NG, 128), lambda i: (0, 0)),
        ],
        out_shape=[
            jax.ShapeDtypeStruct((N, D), _f32),
            jax.ShapeDtypeStruct((N, NG), _f32),
            jax.ShapeDtypeStruct((NG, 128), _f32),
        ],
    )(x2, i2, emb_pad, wn, bn)


def _msg_body(has_v, *refs):
    if has_v:
        (g_ref, e_ref, we_ref, be_ref, w1_ref, b1_ref, wg_ref, bg_ref,
         wv_ref, mlo_ref, mhi_ref, mv_ref) = refs
    else:
        (g_ref, e_ref, we_ref, be_ref, w1_ref, b1_ref, wg_ref, bg_ref,
         mlo_ref, mhi_ref, mv_ref) = refs
    if has_v:
        gs = g_ref[:, :D]
        gv = g_ref[:, D:D + PV]
    else:
        gs = g_ref[...]
        gv = None
    d = e_ref[:, 3:4]
    mu = (lax.broadcasted_iota(jnp.int32, (BE, 128), 1).astype(_f32)
          * (1.0 / (RBF - 1)))
    rbf = jnp.exp(-10.0 * (d - mu) ** 2)
    es = (jnp.dot(rbf, we_ref[...], preferred_element_type=_f32)
          + be_ref[...])
    h = gs * es
    m = _silu(jnp.dot(h, w1_ref[...], preferred_element_type=_f32)
              + b1_ref[...])
    gate = (jnp.dot(m, wg_ref[...], preferred_element_type=_f32)
            + bg_ref[...])
    ev = e_ref[...]
    mvv = jnp.concatenate(
        [gate * ev[:, c:c + 1] for c in range(3)], axis=1)
    if has_v:
        mvv = mvv + jnp.dot(gv, wv_ref[...], preferred_element_type=_f32)
    mlo_ref[...] = m[:, :128]
    mhi_ref[...] = m[:, 128:]
    mv_ref[...] = jnp.concatenate(
        [mvv, jnp.zeros((BE, 128 - PV), _f32)], axis=1)


def _messages(g, e3, we_pad, be, w1, b1, wg_pad, bg_pad, wv_bd, offb, nb):
    has_v = wv_bd is not None
    gw = FW if has_v else D
    in_specs = [
        pl.BlockSpec((BE, gw), lambda i: (i, 0)),
        pl.BlockSpec((BE, 4), lambda i: (i + offb, 0)),
        pl.BlockSpec((128, D), lambda i: (0, 0)),
        pl.BlockSpec((1, D), lambda i: (0, 0)),
        pl.BlockSpec((D, D), lambda i: (0, 0)),
        pl.BlockSpec((1, D), lambda i: (0, 0)),
        pl.BlockSpec((D, VP), lambda i: (0, 0)),
        pl.BlockSpec((1, VP), lambda i: (0, 0)),
    ]
    args = [g, e3, we_pad, be, w1, b1, wg_pad, bg_pad]
    if has_v:
        in_specs.append(pl.BlockSpec((PV, PV), lambda i: (0, 0)))
        args.append(wv_bd)
    return pl.pallas_call(
        functools.partial(_msg_body, has_v),
        grid=(nb,),
        in_specs=in_specs,
        out_specs=[
            pl.BlockSpec((BE, 128), lambda i: (i, 0)),
            pl.BlockSpec((BE, 128), lambda i: (i, 0)),
            pl.BlockSpec((BE, 128), lambda i: (i, 0)),
        ],
        out_shape=[
            jax.ShapeDtypeStruct((nb * BE, 128), _f32),
            jax.ShapeDtypeStruct((nb * BE, 128), _f32),
            jax.ShapeDtypeStruct((nb * BE, 128), _f32),
        ],
    )(*args)


def _upd_body(has_v, *refs):
    if has_v:
        (alo0, alo1, ahi0, ahi1, av0, av1, av2, av3, xsv_ref, oh_ref,
         wu_ref, bu_ref, wmix_ref, wg_ref, xsv_o, gs_o, gv_o) = refs
        xs = xsv_ref[:, :D]
        xv = xsv_ref[:, D:D + PV]
    else:
        (alo0, alo1, ahi0, ahi1, av0, av1, av2, av3, xsv_ref, oh_ref,
         wu_ref, bu_ref, wg_ref, xsv_o, gs_o, gv_o) = refs
        xs = xsv_ref[...]
        xv = None
    pid = pl.program_id(0)
    agg = jnp.concatenate([alo0[...] + alo1[...], ahi0[...] + ahi1[...]],
                          axis=1)
    u = _silu(jnp.dot(agg, wu_ref[...], preferred_element_type=_f32)
              + bu_ref[...])
    xs_n = xs + u
    xv_n = ((av0[:, :PV] + av1[:, :PV])
            + (av2[:, :PV] + av3[:, :PV]))
    if has_v:
        xv_n = xv_n + jnp.dot(xv, wmix_ref[...], preferred_element_type=_f32)
    xsv_o[...] = jnp.concatenate(
        [xs_n, xv_n, jnp.zeros((BN, FW - D - PV), _f32)], axis=1)
    oh = oh_ref[...]
    gsp = lax.dot_general(oh, xs_n, (((0,), (0,)), ((), ())),
                          preferred_element_type=_f32)
    gvz = jnp.dot(xv_n, wg_ref[...], preferred_element_type=_f32)
    gvp = lax.dot_general(oh, gvz, (((0,), (0,)), ((), ())),
                          preferred_element_type=_f32)

    @pl.when(pid == 0)
    def _():
        gs_o[...] = gsp
        gv_o[...] = gvp

    @pl.when(pid > 0)
    def _():
        gs_o[...] += gsp
        gv_o[...] += gvp


def _update(aggs, xsv, oh, wu, bu, wmix_bd, wg_bd):
    has_v = wmix_bd is not None
    xw = FW if has_v else D
    in_specs = [pl.BlockSpec((BN, 128), lambda i: (i, 0))
                for _ in range(8)]
    in_specs += [
        pl.BlockSpec((BN, xw), lambda i: (i, 0)),
        pl.BlockSpec((BN, NG), lambda i: (i, 0)),
        pl.BlockSpec((D, D), lambda i: (0, 0)),
        pl.BlockSpec((1, D), lambda i: (0, 0)),
    ]
    args = list(aggs) + [xsv, oh, wu, bu]
    if has_v:
        in_specs.append(pl.BlockSpec((PV, PV), lambda i: (0, 0)))
        args.append(wmix_bd)
    in_specs.append(pl.BlockSpec((PV, PV), lambda i: (0, 0)))
    args.append(wg_bd)
    return pl.pallas_call(
        functools.partial(_upd_body, has_v),
        grid=(N // BN,),
        in_specs=in_specs,
        out_specs=[
            pl.BlockSpec((BN, FW), lambda i: (i, 0)),
            pl.BlockSpec((NG, D), lambda i: (0, 0)),
            pl.BlockSpec((NG, PV), lambda i: (0, 0)),
        ],
        out_shape=[
            jax.ShapeDtypeStruct((N, FW), _f32),
            jax.ShapeDtypeStruct((NG, D), _f32),
            jax.ShapeDtypeStruct((NG, PV), _f32),
        ],
    )(*args)


def _final_body(*refs):
    cnt_ref = refs[0]
    gs_refs = refs[1:5]
    gv_refs = refs[5:9]
    ws_refs = refs[9:13]
    bs_refs = refs[13:17]
    us_ref, uv_ref = refs[17], refs[18]
    inv = 1.0 / jnp.maximum(cnt_ref[...][:, 0:1], 1.0)
    us = jnp.zeros((NG, 128), _f32)
    uv = jnp.zeros((NG, PV), _f32)
    for l in range(4):
        gs = gs_refs[l][...] * inv
        us = us + (jnp.dot(gs, ws_refs[l][...], preferred_element_type=_f32)
                   + bs_refs[l][...])
        uv = uv + gv_refs[l][...] * inv
    us_ref[...] = us
    uv_ref[...] = uv


def _finalize(cnt, gs_l, gv_l, ws_l, bs_l):
    return pl.pallas_call(
        _final_body,
        out_shape=[
            jax.ShapeDtypeStruct((NG, 128), _f32),
            jax.ShapeDtypeStruct((NG, PV), _f32),
        ],
    )(cnt, *gs_l, *gv_l, *ws_l, *bs_l)


@functools.cache
def _sc_mesh():
    return plsc.VectorSubcoreMesh(core_axis_name="c", subcore_axis_name="s")


def _sc_gather(table, src, width, q0, nq):
    nt0 = (nq + 31) // 32
    NT = nt0 + (nt0 % 2)

    @functools.partial(
        pl.kernel,
        out_type=jax.ShapeDtypeStruct((nq * CH, width), _f32),
        mesh=_sc_mesh(),
        scratch_types=[pltpu.VMEM((CH,), jnp.int32),
                       pltpu.VMEM((CH,), jnp.int32),
                       pltpu.VMEM((CH, width), _f32),
                       pltpu.VMEM((CH, width), _f32),
                       pltpu.SemaphoreType.DMA,
                       pltpu.SemaphoreType.DMA,
                       pltpu.SemaphoreType.DMA,
                       pltpu.SemaphoreType.DMA],
    )
    def k(tab_h, src_h, out_h, idx0, idx1, rows0, rows1, g0, g1, w0, w1):
        wid = lax.axis_index("s") * 2 + lax.axis_index("c")

        def b_of(t):
            return jnp.minimum(wid + t * 32, nq - 1) * CH

        def start_gather(t, idx, rows, gsem):
            pltpu.sync_copy(src_h.at[pl.ds(q0 * CH + b_of(t), CH)], idx)
            pltpu.async_copy(tab_h.at[idx], rows, gsem)

        start_gather(0, idx0, rows0, g0)
        start_gather(1, idx1, rows1, g1)

        @pl.loop(0, NT, step=2)
        def _(t):
            for off, idx, rows, gsem, wsem in ((0, idx0, rows0, g0, w0),
                                               (1, idx1, rows1, g1, w1)):
                tt = t + off
                b = b_of(tt)
                pltpu.make_async_copy(tab_h.at[idx], rows, gsem).wait()
                pltpu.async_copy(rows, out_h.at[pl.ds(b, CH)], wsem)
            for off, idx, rows, gsem, wsem in ((0, idx0, rows0, g0, w0),
                                               (1, idx1, rows1, g1, w1)):
                tt = t + off
                pltpu.make_async_copy(
                    rows, out_h.at[pl.ds(b_of(tt), CH)], wsem).wait()

                @pl.when(tt + 2 < NT)
                def _():
                    start_gather(tt + 2, idx, rows, gsem)

    return k(table, src)


def _sc_scatter_m(mlo, mhi, dst, q0, nq):
    nt0 = (nq + 15) // 16
    NT = nt0 + (nt0 % 2)

    @functools.partial(
        pl.kernel,
        out_type=(jax.ShapeDtypeStruct((NP, 128), _f32),
                  jax.ShapeDtypeStruct((NP, 128), _f32)),
        mesh=_sc_mesh(),
        scratch_types=[pltpu.VMEM((CH,), jnp.int32),
                       pltpu.VMEM((CH,), jnp.int32),
                       pltpu.VMEM((CH, 128), _f32),
                       pltpu.VMEM((CH, 128), _f32),
                       pltpu.VMEM_SHARED((NP, 128), _f32),
                       pltpu.SemaphoreType.DMA,
                       pltpu.SemaphoreType.DMA,
                       pltpu.SemaphoreType.DMA,
                       pltpu.SemaphoreType.DMA],
    )
    def k(mlo_h, mhi_h, dst_h, alo_h, ahi_h,
          idx0, idx1, rows0, rows1, acc_s, l0, l1, s0, s1):
        c = lax.axis_index("c")
        s = lax.axis_index("s")

        @pl.loop(0, CH)
        def _(r):
            @pl.loop(0, 128, step=16)
            def _(l):
                rows0[r, pl.ds(l, 16)] = jnp.zeros((16,), _f32)

        row0 = s * (NP // 16)

        @pl.loop(0, (NP // 16) // CH)
        def _(z):
            pltpu.sync_copy(rows0, acc_s.at[pl.ds(row0 + z * CH, CH)])

        plsc.subcore_barrier()

        def prep_and_load(t, idx, rows, lsem):
            q = s + t * 16
            b = jnp.minimum(q, nq - 1) * CH
            pltpu.sync_copy(dst_h.at[pl.ds(q0 * CH + b, CH)], idx)

            @pl.when(q >= nq)
            def _():
                @pl.loop(0, CH, step=16)
                def _(j):
                    idx[pl.ds(j, 16)] = jnp.full((16,), N, jnp.int32)

            @pl.when(c == 0)
            def _():
                pltpu.sync_copy(mlo_h.at[pl.ds(b, CH)], rows)

            @pl.when(c == 1)
            def _():
                pltpu.sync_copy(mhi_h.at[pl.ds(b, CH)], rows)

        prep_and_load(0, idx0, rows0, l0)
        prep_and_load(1, idx1, rows1, l1)

        @pl.loop(0, NT, step=2)
        def _(t):
            pltpu.async_copy(rows0, acc_s.at[idx0], s0, add=True)
            pltpu.async_copy(rows1, acc_s.at[idx1], s1, add=True)
            for off, idx, rows, lsem, ssem in ((0, idx0, rows0, l0, s0),
                                               (1, idx1, rows1, l1, s1)):
                tt = t + off
                pltpu.make_async_copy(rows, acc_s.at[idx], ssem).wait()

                @pl.when(tt + 2 < NT)
                def _():
                    prep_and_load(tt + 2, idx, rows, lsem)

        plsc.subcore_barrier()

        @pl.loop(0, (NP // 16) // CH)
        def _(z):
            r0 = row0 + z * CH

            @pl.when(c == 0)
            def _():
                pltpu.sync_copy(acc_s.at[pl.ds(r0, CH)],
                                alo_h.at[pl.ds(r0, CH)])

            @pl.when(c == 1)
            def _():
                pltpu.sync_copy(acc_s.at[pl.ds(r0, CH)],
                                ahi_h.at[pl.ds(r0, CH)])

    return k(mlo, mhi, dst)


def _sc_scatter_v(mv, dst, q0, nq):
    nqc = (nq + 1) // 2
    ntc0 = (nqc + 15) // 16
    NTC = ntc0 + (ntc0 % 2)

    @functools.partial(
        pl.kernel,
        out_type=(jax.ShapeDtypeStruct((NP, 128), _f32),
                  jax.ShapeDtypeStruct((NP, 128), _f32)),
        mesh=_sc_mesh(),
        scratch_types=[pltpu.VMEM((CH,), jnp.int32),
                       pltpu.VMEM((CH,), jnp.int32),
                       pltpu.VMEM((CH, 128), _f32),
                       pltpu.VMEM((CH, 128), _f32),
                       pltpu.VMEM_SHARED((NP, 128), _f32),
                       pltpu.SemaphoreType.DMA,
                       pltpu.SemaphoreType.DMA,
                       pltpu.SemaphoreType.DMA,
                       pltpu.SemaphoreType.DMA],
    )
    def k(mv_h, dst_h, av0_h, av1_h,
          idx0, idx1, rows0, rows1, acc_s, l0, l1, s0, s1):
        c = lax.axis_index("c")
        s = lax.axis_index("s")

        @pl.loop(0, CH)
        def _(r):
            @pl.loop(0, 128, step=16)
            def _(l):
                rows0[r, pl.ds(l, 16)] = jnp.zeros((16,), _f32)

        row0 = s * (NP // 16)

        @pl.loop(0, (NP // 16) // CH)
        def _(z):
            pltpu.sync_copy(rows0, acc_s.at[pl.ds(row0 + z * CH, CH)])

        plsc.subcore_barrier()

        qbase = c * nqc
        qend = jnp.minimum(qbase + nqc, nq)

        def prep_and_load(t, idx, rows, lsem):
            q = qbase + s + t * 16
            b = jnp.minimum(q, qend - 1) * CH
            pltpu.sync_copy(dst_h.at[pl.ds(q0 * CH + b, CH)], idx)

            @pl.when(q >= qend)
            def _():
                @pl.loop(0, CH, step=16)
                def _(j):
                    idx[pl.ds(j, 16)] = jnp.full((16,), N, jnp.int32)

            pltpu.sync_copy(mv_h.at[pl.ds(b, CH)], rows)

        prep_and_load(0, idx0, rows0, l0)
        prep_and_load(1, idx1, rows1, l1)

        @pl.loop(0, NTC, step=2)
        def _(t):
            pltpu.async_copy(rows0, acc_s.at[idx0], s0, add=True)
            pltpu.async_copy(rows1, acc_s.at[idx1], s1, add=True)
            for off, idx, rows, lsem, ssem in ((0, idx0, rows0, l0, s0),
                                               (1, idx1, rows1, l1, s1)):
                tt = t + off
                pltpu.make_async_copy(rows, acc_s.at[idx], ssem).wait()

                @pl.when(tt + 2 < NTC)
                def _():
                    prep_and_load(tt + 2, idx, rows, lsem)

        plsc.subcore_barrier()

        @pl.loop(0, (NP // 16) // CH)
        def _(z):
            r0 = row0 + z * CH

            @pl.when(c == 0)
            def _():
                pltpu.sync_copy(acc_s.at[pl.ds(r0, CH)],
                                av0_h.at[pl.ds(r0, CH)])

            @pl.when(c == 1)
            def _():
                pltpu.sync_copy(acc_s.at[pl.ds(r0, CH)],
                                av1_h.at[pl.ds(r0, CH)])

    return k(mv, dst)


def _block_diag(w, vi, vo):
    out = jnp.zeros((PV, PV), _f32)
    for ci in range(3):
        out = out.at[ci * VP:ci * VP + vi, ci * VP:ci * VP + vo].set(w)
    return out


def kernel(x, a, e, i, params):
    src, dst = a[0], a[1]

    we_pad = jnp.zeros((128, D), _f32).at[:RBF].set(params["dense_e"]["W"])
    be = params["dense_e"]["b"].reshape(1, D)
    emb_pad = jnp.zeros((128, D), _f32).at[:NELEM].set(params["emb"])
    wn = params["dense_n"]["W"]
    bn = params["dense_n"]["b"].reshape(1, D)

    x_s, oh, cnt = _init_nodes(x, i.reshape(N, 1).astype(jnp.int32),
                               emb_pad, wn, bn)

    xsv = x_s
    gs_l, gv_l, ws_l, bs_l = [], [], [], []
    for li, ((vi, vo), lp, gp) in enumerate(
            zip(VIVO, params["mpnn"], params["glob"])):
        wg_pad = jnp.zeros((D, VP), _f32).at[:, :vo].set(lp["Wg"]["W"])
        bg_pad = jnp.zeros((1, VP), _f32).at[0, :vo].set(lp["Wg"]["b"])
        wv_bd = _block_diag(lp["Wv"], vi, vo) if li > 0 else None
        wmix_bd = _block_diag(lp["Wmix"], vi, vo) if li > 0 else None
        wgg_bd = _block_diag(gp["Wg"], vo, 3)
        ws_pad = jnp.zeros((D, 128), _f32).at[:, :3].set(gp["Ws"]["W"])
        bs_pad = jnp.zeros((1, 128), _f32).at[0, :3].set(gp["Ws"]["b"])

        width = D if li == 0 else FW
        hq = NCHUNK // 2
        hb = (hq * CH) // BE
        aggs = []
        for h in range(2):
            g = _sc_gather(xsv, src, width, h * hq, hq)
            m_lo, m_hi, m_v = _messages(g, e, we_pad, be,
                                        lp["W1"]["W"],
                                        lp["W1"]["b"].reshape(1, D),
                                        wg_pad, bg_pad, wv_bd,
                                        h * hb, hb)
            a_lo, a_hi = _sc_scatter_m(m_lo, m_hi, dst, h * hq, hq)
            a_v0, a_v1 = _sc_scatter_v(m_v, dst, h * hq, hq)
            aggs.append((a_lo[:N], a_hi[:N], a_v0[:N], a_v1[:N]))

        (al0, ah0, v00, v01), (al1, ah1, v10, v11) = aggs
        xsv, gs, gv = _update((al0, al1, ah0, ah1, v00, v01, v10, v11),
                              xsv, oh,
                              lp["Wu"]["W"],
                              lp["Wu"]["b"].reshape(1, D),
                              wmix_bd, wgg_bd)
        gs_l.append(gs)
        gv_l.append(gv)
        ws_l.append(ws_pad)
        bs_l.append(bs_pad)

    us, uv = _finalize(cnt, gs_l, gv_l, ws_l, bs_l)
    u_s = us[:, :3]
    u_v = uv.reshape(NG, 3, VP)[:, :, :3]
    return jnp.concatenate([u_s[:, :, None], u_v], axis=-1)


# speedup vs baseline: 1.1684x; 1.0018x over previous
import functools

import jax
import jax.numpy as jnp
from jax import lax
from jax.experimental import pallas as pl
from jax.experimental.pallas import tpu as pltpu
from jax.experimental.pallas import tpu_sc as plsc

N = 10000
E = 160000
D = 256
NG = 64
NELEM = 84
RBF = 10
VIVO = [(3, 6), (6, 9), (9, 6), (6, 3)]

VP = 16
PV = 3 * VP
FW = 384
NP = 10240
CH = 128
NCHUNK = E // CH
BE = 1600
BN = 1000

_f32 = jnp.float32


def _silu(x):
    return x * jax.lax.logistic(x)


def _init_body(x_ref, i_ref, emb_ref, wn_ref, bn_ref, xs_ref, oh_ref, cnt_ref):
    pid = pl.program_id(0)
    xv = x_ref[...]
    lane128 = lax.broadcasted_iota(jnp.int32, (BN, 128), 1)
    oh_x = (xv == lane128).astype(_f32)
    emb_rows = jnp.dot(oh_x, emb_ref[...], preferred_element_type=_f32)
    xs_ref[...] = (jnp.dot(emb_rows, wn_ref[...], preferred_element_type=_f32)
                   + bn_ref[...])
    iv = i_ref[...]
    lane64 = lax.broadcasted_iota(jnp.int32, (BN, NG), 1)
    oh = (iv == lane64).astype(_f32)
    oh_ref[...] = oh
    ones = jnp.ones((BN, 128), _f32)
    cpart = lax.dot_general(oh, ones, (((0,), (0,)), ((), ())),
                            preferred_element_type=_f32)

    @pl.when(pid == 0)
    def _():
        cnt_ref[...] = cpart

    @pl.when(pid > 0)
    def _():
        cnt_ref[...] += cpart


def _init_nodes(x2, i2, emb_pad, wn, bn):
    return pl.pallas_call(
        _init_body,
        grid=(N // BN,),
        in_specs=[
            pl.BlockSpec((BN, 1), lambda i: (i, 0)),
            pl.BlockSpec((BN, 1), lambda i: (i, 0)),
            pl.BlockSpec((128, D), lambda i: (0, 0)),
            pl.BlockSpec((D, D), lambda i: (0, 0)),
            pl.BlockSpec((1, D), lambda i: (0, 0)),
        ],
        out_specs=[
            pl.BlockSpec((BN, D), lambda i: (i, 0)),
            pl.BlockSpec((BN, NG), lambda i: (i, 0)),
            pl.BlockSpec((NG, 128), lambda i: (0, 0)),
        ],
        out_shape=[
            jax.ShapeDtypeStruct((N, D), _f32),
            jax.ShapeDtypeStruct((N, NG), _f32),
            jax.ShapeDtypeStruct((NG, 128), _f32),
        ],
    )(x2, i2, emb_pad, wn, bn)


def _msg_body(has_v, *refs):
    if has_v:
        (g_ref, e_ref, we_ref, be_ref, w1_ref, b1_ref, wg_ref, bg_ref,
         wv_ref, mlo_ref, mhi_ref, mv_ref) = refs
    else:
        (g_ref, e_ref, we_ref, be_ref, w1_ref, b1_ref, wg_ref, bg_ref,
         mlo_ref, mhi_ref, mv_ref) = refs
    if has_v:
        gs = g_ref[:, :D]
        gv = g_ref[:, D:D + PV]
    else:
        gs = g_ref[...]
        gv = None
    d = e_ref[:, 3:4]
    mu = (lax.broadcasted_iota(jnp.int32, (BE, 128), 1).astype(_f32)
          * (1.0 / (RBF - 1)))
    rbf = jnp.exp(-10.0 * (d - mu) ** 2)
    es = (jnp.dot(rbf, we_ref[...], preferred_element_type=_f32)
          + be_ref[...])
    h = gs * es
    m = _silu(jnp.dot(h, w1_ref[...], preferred_element_type=_f32)
              + b1_ref[...])
    gate = (jnp.dot(m, wg_ref[...], preferred_element_type=_f32)
            + bg_ref[...])
    ev = e_ref[...]
    mvv = jnp.concatenate(
        [gate * ev[:, c:c + 1] for c in range(3)], axis=1)
    if has_v:
        mvv = mvv + jnp.dot(gv, wv_ref[...], preferred_element_type=_f32)
    mlo_ref[...] = m[:, :128]
    mhi_ref[...] = m[:, 128:]
    mv_ref[...] = jnp.concatenate(
        [mvv, jnp.zeros((BE, 128 - PV), _f32)], axis=1)


def _messages(g, e3, we_pad, be, w1, b1, wg_pad, bg_pad, wv_bd, offb, nb):
    has_v = wv_bd is not None
    gw = FW if has_v else D
    in_specs = [
        pl.BlockSpec((BE, gw), lambda i: (i, 0)),
        pl.BlockSpec((BE, 4), lambda i: (i + offb, 0)),
        pl.BlockSpec((128, D), lambda i: (0, 0)),
        pl.BlockSpec((1, D), lambda i: (0, 0)),
        pl.BlockSpec((D, D), lambda i: (0, 0)),
        pl.BlockSpec((1, D), lambda i: (0, 0)),
        pl.BlockSpec((D, VP), lambda i: (0, 0)),
        pl.BlockSpec((1, VP), lambda i: (0, 0)),
    ]
    args = [g, e3, we_pad, be, w1, b1, wg_pad, bg_pad]
    if has_v:
        in_specs.append(pl.BlockSpec((PV, PV), lambda i: (0, 0)))
        args.append(wv_bd)
    return pl.pallas_call(
        functools.partial(_msg_body, has_v),
        grid=(nb,),
        in_specs=in_specs,
        out_specs=[
            pl.BlockSpec((BE, 128), lambda i: (i, 0)),
            pl.BlockSpec((BE, 128), lambda i: (i, 0)),
            pl.BlockSpec((BE, 128), lambda i: (i, 0)),
        ],
        out_shape=[
            jax.ShapeDtypeStruct((nb * BE, 128), _f32),
            jax.ShapeDtypeStruct((nb * BE, 128), _f32),
            jax.ShapeDtypeStruct((nb * BE, 128), _f32),
        ],
    )(*args)


def _upd_body(has_v, *refs):
    if has_v:
        (alo0, alo1, ahi0, ahi1, av0, av1, av2, av3, xsv_ref, oh_ref,
         wu_ref, bu_ref, wmix_ref, wg_ref, xsv_o, gs_o, gv_o) = refs
        xs = xsv_ref[:, :D]
        xv = xsv_ref[:, D:D + PV]
    else:
        (alo0, alo1, ahi0, ahi1, av0, av1, av2, av3, xsv_ref, oh_ref,
         wu_ref, bu_ref, wg_ref, xsv_o, gs_o, gv_o) = refs
        xs = xsv_ref[...]
        xv = None
    pid = pl.program_id(0)
    agg = jnp.concatenate([alo0[...] + alo1[...], ahi0[...] + ahi1[...]],
                          axis=1)
    u = _silu(jnp.dot(agg, wu_ref[...], preferred_element_type=_f32)
              + bu_ref[...])
    xs_n = xs + u
    xv_n = ((av0[:, :PV] + av1[:, :PV])
            + (av2[:, :PV] + av3[:, :PV]))
    if has_v:
        xv_n = xv_n + jnp.dot(xv, wmix_ref[...], preferred_element_type=_f32)
    xsv_o[...] = jnp.concatenate(
        [xs_n, xv_n, jnp.zeros((BN, FW - D - PV), _f32)], axis=1)
    oh = oh_ref[...]
    gsp = lax.dot_general(oh, xs_n, (((0,), (0,)), ((), ())),
                          preferred_element_type=_f32)
    gvz = jnp.dot(xv_n, wg_ref[...], preferred_element_type=_f32)
    gvp = lax.dot_general(oh, gvz, (((0,), (0,)), ((), ())),
                          preferred_element_type=_f32)

    @pl.when(pid == 0)
    def _():
        gs_o[...] = gsp
        gv_o[...] = gvp

    @pl.when(pid > 0)
    def _():
        gs_o[...] += gsp
        gv_o[...] += gvp


def _update(aggs, xsv, oh, wu, bu, wmix_bd, wg_bd):
    has_v = wmix_bd is not None
    xw = FW if has_v else D
    in_specs = [pl.BlockSpec((BN, 128), lambda i: (i, 0))
                for _ in range(8)]
    in_specs += [
        pl.BlockSpec((BN, xw), lambda i: (i, 0)),
        pl.BlockSpec((BN, NG), lambda i: (i, 0)),
        pl.BlockSpec((D, D), lambda i: (0, 0)),
        pl.BlockSpec((1, D), lambda i: (0, 0)),
    ]
    args = list(aggs) + [xsv, oh, wu, bu]
    if has_v:
        in_specs.append(pl.BlockSpec((PV, PV), lambda i: (0, 0)))
        args.append(wmix_bd)
    in_specs.append(pl.BlockSpec((PV, PV), lambda i: (0, 0)))
    args.append(wg_bd)
    return pl.pallas_call(
        functools.partial(_upd_body, has_v),
        grid=(N // BN,),
        in_specs=in_specs,
        out_specs=[
            pl.BlockSpec((BN, FW), lambda i: (i, 0)),
            pl.BlockSpec((NG, D), lambda i: (0, 0)),
            pl.BlockSpec((NG, PV), lambda i: (0, 0)),
        ],
        out_shape=[
            jax.ShapeDtypeStruct((N, FW), _f32),
            jax.ShapeDtypeStruct((NG, D), _f32),
            jax.ShapeDtypeStruct((NG, PV), _f32),
        ],
    )(*args)


def _final_body(*refs):
    cnt_ref = refs[0]
    gs_refs = refs[1:5]
    gv_refs = refs[5:9]
    ws_refs = refs[9:13]
    bs_refs = refs[13:17]
    us_ref, uv_ref = refs[17], refs[18]
    inv = 1.0 / jnp.maximum(cnt_ref[...][:, 0:1], 1.0)
    us = jnp.zeros((NG, 128), _f32)
    uv = jnp.zeros((NG, PV), _f32)
    for l in range(4):
        gs = gs_refs[l][...] * inv
        us = us + (jnp.dot(gs, ws_refs[l][...], preferred_element_type=_f32)
                   + bs_refs[l][...])
        uv = uv + gv_refs[l][...] * inv
    us_ref[...] = us
    uv_ref[...] = uv


def _finalize(cnt, gs_l, gv_l, ws_l, bs_l):
    return pl.pallas_call(
        _final_body,
        out_shape=[
            jax.ShapeDtypeStruct((NG, 128), _f32),
            jax.ShapeDtypeStruct((NG, PV), _f32),
        ],
    )(cnt, *gs_l, *gv_l, *ws_l, *bs_l)


@functools.cache
def _sc_mesh():
    return plsc.VectorSubcoreMesh(core_axis_name="c", subcore_axis_name="s")


def _sc_gather(table, src, width, q0, nq):
    nt0 = (nq + 31) // 32
    NT = nt0 + (nt0 % 2)

    @functools.partial(
        pl.kernel,
        out_type=jax.ShapeDtypeStruct((nq * CH, width), _f32),
        mesh=_sc_mesh(),
        scratch_types=[pltpu.VMEM((CH,), jnp.int32),
                       pltpu.VMEM((CH,), jnp.int32),
                       pltpu.VMEM((CH, width), _f32),
                       pltpu.VMEM((CH, width), _f32),
                       pltpu.SemaphoreType.DMA,
                       pltpu.SemaphoreType.DMA,
                       pltpu.SemaphoreType.DMA,
                       pltpu.SemaphoreType.DMA],
    )
    def k(tab_h, src_h, out_h, idx0, idx1, rows0, rows1, g0, g1, w0, w1):
        wid = lax.axis_index("s") * 2 + lax.axis_index("c")

        def b_of(t):
            return jnp.minimum(wid + t * 32, nq - 1) * CH

        def start_gather(t, idx, rows, gsem):
            pltpu.sync_copy(src_h.at[pl.ds(q0 * CH + b_of(t), CH)], idx)
            pltpu.async_copy(tab_h.at[idx], rows, gsem)

        start_gather(0, idx0, rows0, g0)
        start_gather(1, idx1, rows1, g1)

        @pl.loop(0, NT, step=2)
        def _(t):
            for off, idx, rows, gsem, wsem in ((0, idx0, rows0, g0, w0),
                                               (1, idx1, rows1, g1, w1)):
                tt = t + off
                b = b_of(tt)
                pltpu.make_async_copy(tab_h.at[idx], rows, gsem).wait()
                pltpu.async_copy(rows, out_h.at[pl.ds(b, CH)], wsem)
            for off, idx, rows, gsem, wsem in ((0, idx0, rows0, g0, w0),
                                               (1, idx1, rows1, g1, w1)):
                tt = t + off
                pltpu.make_async_copy(
                    rows, out_h.at[pl.ds(b_of(tt), CH)], wsem).wait()

                @pl.when(tt + 2 < NT)
                def _():
                    start_gather(tt + 2, idx, rows, gsem)

    return k(table, src)


def _sc_scatter_m(mlo, mhi, dst, q0, nq):
    nt0 = (nq + 15) // 16
    NT = nt0 + (nt0 % 2)

    @functools.partial(
        pl.kernel,
        out_type=(jax.ShapeDtypeStruct((NP, 128), _f32),
                  jax.ShapeDtypeStruct((NP, 128), _f32)),
        mesh=_sc_mesh(),
        scratch_types=[pltpu.VMEM((CH,), jnp.int32),
                       pltpu.VMEM((CH,), jnp.int32),
                       pltpu.VMEM((CH, 128), _f32),
                       pltpu.VMEM((CH, 128), _f32),
                       pltpu.VMEM_SHARED((NP, 128), _f32),
                       pltpu.SemaphoreType.DMA,
                       pltpu.SemaphoreType.DMA,
                       pltpu.SemaphoreType.DMA,
                       pltpu.SemaphoreType.DMA],
    )
    def k(mlo_h, mhi_h, dst_h, alo_h, ahi_h,
          idx0, idx1, rows0, rows1, acc_s, l0, l1, s0, s1):
        c = lax.axis_index("c")
        s = lax.axis_index("s")

        @pl.loop(0, CH)
        def _(r):
            @pl.loop(0, 128, step=16)
            def _(l):
                rows0[r, pl.ds(l, 16)] = jnp.zeros((16,), _f32)

        row0 = s * (NP // 16)

        @pl.loop(0, (NP // 16) // CH)
        def _(z):
            pltpu.sync_copy(rows0, acc_s.at[pl.ds(row0 + z * CH, CH)])

        plsc.subcore_barrier()

        def prep_and_load(t, idx, rows, lsem):
            q = s + t * 16
            b = jnp.minimum(q, nq - 1) * CH
            pltpu.sync_copy(dst_h.at[pl.ds(q0 * CH + b, CH)], idx)

            @pl.when(q >= nq)
            def _():
                @pl.loop(0, CH, step=16)
                def _(j):
                    idx[pl.ds(j, 16)] = jnp.full((16,), N, jnp.int32)

            @pl.when(c == 0)
            def _():
                pltpu.sync_copy(mlo_h.at[pl.ds(b, CH)], rows)

            @pl.when(c == 1)
            def _():
                pltpu.sync_copy(mhi_h.at[pl.ds(b, CH)], rows)

        prep_and_load(0, idx0, rows0, l0)
        prep_and_load(1, idx1, rows1, l1)

        @pl.loop(0, NT, step=2)
        def _(t):
            pltpu.async_copy(rows0, acc_s.at[idx0], s0, add=True)
            pltpu.async_copy(rows1, acc_s.at[idx1], s1, add=True)
            for off, idx, rows, lsem, ssem in ((0, idx0, rows0, l0, s0),
                                               (1, idx1, rows1, l1, s1)):
                tt = t + off
                pltpu.make_async_copy(rows, acc_s.at[idx], ssem).wait()

                @pl.when(tt + 2 < NT)
                def _():
                    prep_and_load(tt + 2, idx, rows, lsem)

        plsc.subcore_barrier()

        @pl.loop(0, (NP // 16) // CH)
        def _(z):
            r0 = row0 + z * CH

            @pl.when(c == 0)
            def _():
                pltpu.sync_copy(acc_s.at[pl.ds(r0, CH)],
                                alo_h.at[pl.ds(r0, CH)])

            @pl.when(c == 1)
            def _():
                pltpu.sync_copy(acc_s.at[pl.ds(r0, CH)],
                                ahi_h.at[pl.ds(r0, CH)])

    return k(mlo, mhi, dst)


def _sc_scatter_v(mv, dst, q0, nq):
    nqc = (nq + 1) // 2
    ntc0 = (nqc + 15) // 16
    NTC = ntc0 + (ntc0 % 2)

    @functools.partial(
        pl.kernel,
        out_type=(jax.ShapeDtypeStruct((NP, 128), _f32),
                  jax.ShapeDtypeStruct((NP, 128), _f32)),
        mesh=_sc_mesh(),
        scratch_types=[pltpu.VMEM((CH,), jnp.int32),
                       pltpu.VMEM((CH,), jnp.int32),
                       pltpu.VMEM((CH, 128), _f32),
                       pltpu.VMEM((CH, 128), _f32),
                       pltpu.VMEM_SHARED((NP, 128), _f32),
                       pltpu.SemaphoreType.DMA,
                       pltpu.SemaphoreType.DMA,
                       pltpu.SemaphoreType.DMA,
                       pltpu.SemaphoreType.DMA],
    )
    def k(mv_h, dst_h, av0_h, av1_h,
          idx0, idx1, rows0, rows1, acc_s, l0, l1, s0, s1):
        c = lax.axis_index("c")
        s = lax.axis_index("s")

        @pl.loop(0, CH)
        def _(r):
            @pl.loop(0, 128, step=16)
            def _(l):
                rows0[r, pl.ds(l, 16)] = jnp.zeros((16,), _f32)

        row0 = s * (NP // 16)

        @pl.loop(0, (NP // 16) // CH)
        def _(z):
            pltpu.sync_copy(rows0, acc_s.at[pl.ds(row0 + z * CH, CH)])

        plsc.subcore_barrier()

        qbase = c * nqc
        qend = jnp.minimum(qbase + nqc, nq)

        def prep_and_load(t, idx, rows, lsem):
            q = qbase + s + t * 16
            b = jnp.minimum(q, qend - 1) * CH
            pltpu.sync_copy(dst_h.at[pl.ds(q0 * CH + b, CH)], idx)

            @pl.when(q >= qend)
            def _():
                @pl.loop(0, CH, step=16)
                def _(j):
                    idx[pl.ds(j, 16)] = jnp.full((16,), N, jnp.int32)

            pltpu.sync_copy(mv_h.at[pl.ds(b, CH)], rows)

        prep_and_load(0, idx0, rows0, l0)
        prep_and_load(1, idx1, rows1, l1)

        @pl.loop(0, NTC, step=2)
        def _(t):
            pltpu.async_copy(rows0, acc_s.at[idx0], s0, add=True)
            pltpu.async_copy(rows1, acc_s.at[idx1], s1, add=True)
            for off, idx, rows, lsem, ssem in ((0, idx0, rows0, l0, s0),
                                               (1, idx1, rows1, l1, s1)):
                tt = t + off
                pltpu.make_async_copy(rows, acc_s.at[idx], ssem).wait()

                @pl.when(tt + 2 < NTC)
                def _():
                    prep_and_load(tt + 2, idx, rows, lsem)

        plsc.subcore_barrier()

        @pl.loop(0, (NP // 16) // CH)
        def _(z):
            r0 = row0 + z * CH

            @pl.when(c == 0)
            def _():
                pltpu.sync_copy(acc_s.at[pl.ds(r0, CH)],
                                av0_h.at[pl.ds(r0, CH)])

            @pl.when(c == 1)
            def _():
                pltpu.sync_copy(acc_s.at[pl.ds(r0, CH)],
                                av1_h.at[pl.ds(r0, CH)])

    return k(mv, dst)


def _block_diag(w, vi, vo):
    out = jnp.zeros((PV, PV), _f32)
    for ci in range(3):
        out = out.at[ci * VP:ci * VP + vi, ci * VP:ci * VP + vo].set(w)
    return out


def kernel(x, a, e, i, params):
    src, dst = a[0], a[1]

    we_pad = jnp.zeros((128, D), _f32).at[:RBF].set(params["dense_e"]["W"])
    be = params["dense_e"]["b"].reshape(1, D)
    emb_pad = jnp.zeros((128, D), _f32).at[:NELEM].set(params["emb"])
    wn = params["dense_n"]["W"]
    bn = params["dense_n"]["b"].reshape(1, D)

    x_s, oh, cnt = _init_nodes(x, i.reshape(N, 1).astype(jnp.int32),
                               emb_pad, wn, bn)

    xsv = x_s
    gs_l, gv_l, ws_l, bs_l = [], [], [], []
    for li, ((vi, vo), lp, gp) in enumerate(
            zip(VIVO, params["mpnn"], params["glob"])):
        wg_pad = jnp.zeros((D, VP), _f32).at[:, :vo].set(lp["Wg"]["W"])
        bg_pad = jnp.zeros((1, VP), _f32).at[0, :vo].set(lp["Wg"]["b"])
        wv_bd = _block_diag(lp["Wv"], vi, vo) if li > 0 else None
        wmix_bd = _block_diag(lp["Wmix"], vi, vo) if li > 0 else None
        wgg_bd = _block_diag(gp["Wg"], vo, 3)
        ws_pad = jnp.zeros((D, 128), _f32).at[:, :3].set(gp["Ws"]["W"])
        bs_pad = jnp.zeros((1, 128), _f32).at[0, :3].set(gp["Ws"]["b"])

        width = D if li == 0 else FW
        hq = NCHUNK // 2
        hb = (hq * CH) // BE
        aggs = []
        for h in range(2):
            g = _sc_gather(xsv, src, width, h * hq, hq)
            m_lo, m_hi, m_v = _messages(g, e, we_pad, be,
                                        lp["W1"]["W"],
                                        lp["W1"]["b"].reshape(1, D),
                                        wg_pad, bg_pad, wv_bd,
                                        h * hb, hb)
            a_lo, a_hi = _sc_scatter_m(m_lo, m_hi, dst, h * hq, hq)
            a_v0, a_v1 = _sc_scatter_v(m_v, dst, h * hq, hq)
            aggs.append((a_lo[:N], a_hi[:N], a_v0[:N], a_v1[:N]))

        (al0, ah0, v00, v01), (al1, ah1, v10, v11) = aggs
        xsv, gs, gv = _update((al0, al1, ah0, ah1, v00, v01, v10, v11),
                              xsv, oh,
                              lp["Wu"]["W"],
                              lp["Wu"]["b"].reshape(1, D),
                              wmix_bd, wgg_bd)
        gs_l.append(gs)
        gv_l.append(gv)
        ws_l.append(ws_pad)
        bs_l.append(bs_pad)

    us, uv = _finalize(cnt, gs_l, gv_l, ws_l, bs_l)
    u_s = us[:, :3]
    u_v = uv.reshape(NG, 3, VP)[:, :, :3]
    return jnp.concatenate([u_s[:, :, None], u_v], axis=-1)
